# baseline (speedup 1.0000x reference)
import functools

import jax
import jax.numpy as jnp
from jax import lax
from jax.experimental import pallas as pl
from jax.experimental.pallas import tpu as pltpu
from jax.experimental.pallas import tpu_sc as plsc

D = 32
NC = 2
NS = 16
NW = NC * NS
G = 128
NB = 8


@functools.partial(jax.jit, static_argnames=("n_s", "n_b"))
def _gather_t(idx, table, n_s, n_b):
    bt_per_s = n_b // G
    n_blocks = n_s * bt_per_s
    blk_per_w = n_blocks // NW
    idx_per_w = blk_per_w * G
    n_groups = blk_per_w // NB
    mesh = plsc.VectorSubcoreMesh(core_axis_name="c", subcore_axis_name="s")

    @functools.partial(
        pl.kernel,
        mesh=mesh,
        out_type=jax.ShapeDtypeStruct((n_s, D // 8, bt_per_s, 8, G), jnp.float32),
        scratch_types=[
            pltpu.VMEM((idx_per_w,), jnp.int32),
            pltpu.VMEM((NB, G, D), jnp.float32),
            pltpu.VMEM((D // 8, NB, 8, G), jnp.float32),
            [pltpu.SemaphoreType.DMA] * NB,
            pltpu.SemaphoreType.DMA,
        ],
        compiler_params=pltpu.CompilerParams(
            use_tc_tiling_on_sc=False,
            needs_layout_passes=False,
            disable_bounds_checks=True,
        ),
    )
    def emb(idx_hbm, table_hbm, out_hbm, idx_v, rows_v, stg_v, gsems, ssem):
        wid = lax.axis_index("s") * NC + lax.axis_index("c")
        base_blk = wid * blk_per_w
        bases = [lax.iota(jnp.int32, 16) + l * 16 for l in range(G // 16)]

        pltpu.sync_copy(idx_hbm.at[pl.ds(wid * idx_per_w, idx_per_w)], idx_v)

        def fire(local_blk, slot):
            pltpu.async_copy(
                table_hbm.at[idx_v.at[pl.ds(local_blk * G, G)]],
                rows_v.at[slot],
                gsems[slot],
            )

        for u in range(NB):
            fire(u, u)

        def body(sg, carry):
            blk0 = base_blk + sg * NB
            s = blk0 // bt_per_s
            bt0 = blk0 % bt_per_s
            @pl.when(sg > 0)
            def _():
                pltpu.make_async_copy(
                    stg_v, out_hbm.at[0, :, pl.ds(0, NB)], ssem
                ).wait()

            for j in range(NB):
                b = sg * NB + j
                pltpu.make_async_copy(
                    table_hbm.at[idx_v.at[pl.ds(0, G)]], rows_v.at[j], gsems[j]
                ).wait()
                for dd in range(D):
                    col = jnp.full((16,), dd, jnp.int32)
                    vs = [
                        plsc.load_gather(rows_v.at[j], [bases[l], col])
                        for l in range(G // 16)
                    ]
                    for l, v in enumerate(vs):
                        stg_v[dd // 8, j, dd % 8, pl.ds(l * 16, 16)] = v

                @pl.when(b + NB < blk_per_w)
                def _():
                    fire(b + NB, j)

            pltpu.async_copy(stg_v, out_hbm.at[s, :, pl.ds(bt0, NB)], ssem)
            return carry

        lax.fori_loop(0, n_groups, body, 0)

        pltpu.make_async_copy(
            stg_v, out_hbm.at[0, :, pl.ds(0, NB)], ssem
        ).wait()

    return emb(idx, table)


def kernel(x, table):
    b, s = x.shape
    idx = jnp.reshape(jnp.transpose(x), (b * s,)).astype(jnp.int32)
    out5 = _gather_t(idx, table, s, b)
    return jnp.reshape(jnp.transpose(out5, (2, 4, 0, 1, 3)), (b, s, D))


# speedup vs baseline: 1.0888x; 1.0888x over previous
import functools

import jax
import jax.numpy as jnp
from jax import lax
from jax.experimental import pallas as pl
from jax.experimental.pallas import tpu as pltpu
from jax.experimental.pallas import tpu_sc as plsc

D = 32
NC = 2
NS = 16
NW = NC * NS
C = 640
G = 128
NB = 4


@functools.partial(jax.jit, static_argnames=("n_rows",))
def _gather_rows(idx, table, n_rows):
    b_per_w = n_rows // NW
    n_chunks = b_per_w // C
    n_groups = n_chunks // NB
    mesh = plsc.VectorSubcoreMesh(core_axis_name="c", subcore_axis_name="s")

    @functools.partial(
        pl.kernel,
        mesh=mesh,
        out_type=jax.ShapeDtypeStruct((n_rows, D), jnp.float32),
        scratch_types=[
            pltpu.VMEM((NB, C), jnp.int32),
            pltpu.VMEM((NB, C, D), jnp.float32),
            [pltpu.SemaphoreType.DMA] * NB,
            [pltpu.SemaphoreType.DMA] * NB,
        ],
        compiler_params=pltpu.CompilerParams(use_tc_tiling_on_sc=False),
    )
    def emb(idx_hbm, table_hbm, out_hbm, idx_v, rows_v, gsems, ssems):
        wid = lax.axis_index("s") * NC + lax.axis_index("c")
        base = wid * b_per_w

        def fire(chunk, b):
            off = base + chunk * C
            pltpu.sync_copy(idx_hbm.at[pl.ds(off, C)], idx_v.at[b])
            for j in range(C // G):
                pltpu.async_copy(
                    table_hbm.at[idx_v.at[b, pl.ds(j * G, G)]],
                    rows_v.at[b, pl.ds(j * G, G)],
                    gsems[b],
                )

        for b in range(NB):
            fire(b, b)

        def body(g, carry):
            for b in range(NB):
                chunk = g * NB + b
                off = base + chunk * C
                for j in range(C // G):
                    pltpu.make_async_copy(
                        table_hbm.at[idx_v.at[b, pl.ds(j * G, G)]],
                        rows_v.at[b, pl.ds(j * G, G)],
                        gsems[b],
                    ).wait()
                pltpu.async_copy(rows_v.at[b], out_hbm.at[pl.ds(off, C)], ssems[b])

            @pl.when(g < n_groups - 1)
            def _():
                for b in range(NB):
                    pltpu.make_async_copy(
                        rows_v.at[b], out_hbm.at[pl.ds(base, C)], ssems[b]
                    ).wait()
                    fire((g + 1) * NB + b, b)
            return carry

        lax.fori_loop(0, n_groups, body, 0)

        for b in range(NB):
            pltpu.make_async_copy(
                rows_v.at[b], out_hbm.at[pl.ds(base, C)], ssems[b]
            ).wait()

    return emb(idx, table)


def kernel(x, table):
    b, s = x.shape
    n_rows = b * s
    idx = jnp.reshape(x.astype(jnp.int32), (n_rows,))
    out = _gather_rows(idx, table, n_rows)
    return jnp.reshape(out, (b, s, D))


# speedup vs baseline: 1.1455x; 1.0521x over previous
import functools

import jax
import jax.numpy as jnp
from jax import lax
from jax.experimental import pallas as pl
from jax.experimental.pallas import tpu as pltpu
from jax.experimental.pallas import tpu_sc as plsc

D = 32
NC = 2
NS = 16
NW = NC * NS
C = 640
G = 128
NB = 4


@functools.partial(jax.jit, static_argnames=("n_rows",))
def _gather_rows(idx, table, n_rows):
    b_per_w = n_rows // NW
    n_chunks = b_per_w // C
    n_groups = n_chunks // NB
    mesh = plsc.VectorSubcoreMesh(core_axis_name="c", subcore_axis_name="s")

    @functools.partial(
        pl.kernel,
        mesh=mesh,
        out_type=jax.ShapeDtypeStruct((n_rows, D), jnp.float32),
        scratch_types=[
            pltpu.VMEM((NB, C), jnp.int32),
            pltpu.VMEM((NB, C, D), jnp.float32),
            [pltpu.SemaphoreType.DMA] * NB,
            [pltpu.SemaphoreType.DMA] * NB,
        ],
        compiler_params=pltpu.CompilerParams(use_tc_tiling_on_sc=False),
    )
    def emb(idx_hbm, table_hbm, out_hbm, idx_v, rows_v, gsems, ssems):
        wid = lax.axis_index("s") * NC + lax.axis_index("c")
        base = wid * b_per_w

        def fire(chunk, b):
            off = base + chunk * C
            pltpu.sync_copy(idx_hbm.at[pl.ds(off, C)], idx_v.at[b])
            for j in range(C // G):
                pltpu.async_copy(
                    table_hbm.at[idx_v.at[b, pl.ds(j * G, G)]],
                    rows_v.at[b, pl.ds(j * G, G)],
                    gsems[b],
                )

        for b in range(NB):
            fire(b, b)

        def body(g, carry):
            for b in range(NB):
                chunk = g * NB + b
                off = base + chunk * C
                for j in range(C // G):
                    pltpu.make_async_copy(
                        table_hbm.at[idx_v.at[b, pl.ds(j * G, G)]],
                        rows_v.at[b, pl.ds(j * G, G)],
                        gsems[b],
                    ).wait()
                pltpu.async_copy(rows_v.at[b], out_hbm.at[pl.ds(off, C)], ssems[b])

            @pl.when(g < n_groups - 1)
            def _():
                for b in range(NB):
                    pltpu.make_async_copy(
                        rows_v.at[b], out_hbm.at[pl.ds(base, C)], ssems[b]
                    ).wait()
                    fire((g + 1) * NB + b, b)
            return carry

        lax.fori_loop(0, n_groups, body, 0)

        for b in range(NB):
            pltpu.make_async_copy(
                rows_v.at[b], out_hbm.at[pl.ds(base, C)], ssems[b]
            ).wait()

    return emb(idx, table)


def kernel(x, table):
    b, s = x.shape
    n_rows = b * s
    idx = jnp.maximum(
        jnp.reshape(jnp.transpose(x), (n_rows,)), 0
    ).astype(jnp.int32)
    out = _gather_rows(idx, table, n_rows)
    return jnp.transpose(jnp.reshape(out, (s, b, D)), (1, 0, 2))
